# baseline (speedup 1.0000x reference)
import functools
import math

import jax
import jax.numpy as jnp
from jax import lax
from jax.experimental import pallas as pl
from jax.experimental.pallas import tpu as pltpu
from jax.experimental.pallas import tpu_sc as plsc

L = 16
NW = 32
NPW = 320
CHUNK = 160
DEG = 32


def _gather_body(agg_hbm, nb_hbm, out_hbm, idxv, rows, sem):
    wid = lax.axis_index("s") * 2 + lax.axis_index("c")
    for chunk in range(NPW // CHUNK):
        nbase = wid * NPW + chunk * CHUNK
        pltpu.sync_copy(nb_hbm.at[pl.ds(nbase * DEG, CHUNK * DEG)], idxv)
        pltpu.async_copy(agg_hbm.at[idxv], rows, sem).wait()
        pltpu.sync_copy(rows, out_hbm.at[pl.ds(nbase * DEG, CHUNK * DEG)])


def _gather_sc(agg_pad, nb_pad, node_pad):
    f = functools.partial(
        pl.kernel,
        out_type=jax.ShapeDtypeStruct((node_pad * DEG, L), jnp.float32),
        mesh=plsc.VectorSubcoreMesh(core_axis_name="c", subcore_axis_name="s"),
        compiler_params=pltpu.CompilerParams(
            needs_layout_passes=False, use_tc_tiling_on_sc=False),
        scratch_types=[
            pltpu.VMEM((CHUNK * DEG,), jnp.int32),
            pltpu.VMEM((CHUNK * DEG, L), jnp.float32),
            pltpu.SemaphoreType.DMA,
        ],
    )(_gather_body)
    return f(agg_pad, nb_pad)


def _adj_body(a_ref, bt_ref, o_ref):
    x = jnp.dot(a_ref[...], bt_ref[...], preferred_element_type=jnp.float32)
    o_ref[...] = 0.5 * (jnp.tanh(0.5 * x) + 1.0)


def _attn_body(agg_ref, nb_ref, ctx_ref):
    md = DEG * L
    k32 = lax.broadcasted_iota(jnp.int32, (md, DEG), 0)
    m32 = lax.broadcasted_iota(jnp.int32, (md, DEG), 1)
    seg = (jnp.right_shift(k32, 4) == m32).astype(jnp.float32)
    k16 = lax.broadcasted_iota(jnp.int32, (md, L), 0)
    d16 = lax.broadcasted_iota(jnp.int32, (md, L), 1)
    dmat = (jnp.bitwise_and(k16, L - 1) == d16).astype(jnp.float32)

    agg = agg_ref[...]
    nb = nb_ref[...]
    a_exp = jnp.dot(agg, dmat.T, preferred_element_type=jnp.float32)
    s = jnp.dot(a_exp * nb, seg, preferred_element_type=jnp.float32)
    s = s * jnp.float32(1.0 / math.sqrt(L))
    mx = jnp.max(s, axis=-1, keepdims=True)
    e = jnp.exp(s - mx)
    p = e / jnp.sum(e, axis=-1, keepdims=True)
    q = jnp.dot(p, seg.T, preferred_element_type=jnp.float32)
    ctx_ref[...] = jnp.dot(q * nb, dmat, preferred_element_type=jnp.float32)


def _attr_body(nf_ref, ctx_ref, w1_ref, w2_ref, b_ref, o_ref):
    acc = jnp.dot(nf_ref[...], w1_ref[...], preferred_element_type=jnp.float32)
    acc = acc + jnp.dot(ctx_ref[...], w2_ref[...], preferred_element_type=jnp.float32)
    o_ref[...] = jax.nn.sigmoid(acc + b_ref[...])


def kernel(node_feature, agg_feature, nb_id, W, b):
    n = agg_feature.shape[0]
    d = agg_feature.shape[1]
    node_dim = node_feature.shape[1]
    node_pad = NW * NPW
    md = DEG * d

    agg_pad = jnp.pad(agg_feature, ((0, node_pad - n), (0, 0)))
    nb_pad = jnp.pad(nb_id, (0, node_pad * DEG - nb_id.shape[0]))

    nb2 = _gather_sc(agg_pad, nb_pad, node_pad).reshape(node_pad, md)

    BI, BJ = 512, 1024
    adj = pl.pallas_call(
        _adj_body,
        grid=(pl.cdiv(n, BI), pl.cdiv(n, BJ)),
        in_specs=[
            pl.BlockSpec((BI, d), lambda i, j: (i, 0)),
            pl.BlockSpec((d, BJ), lambda i, j: (0, j)),
        ],
        out_specs=pl.BlockSpec((BI, BJ), lambda i, j: (i, j)),
        out_shape=jax.ShapeDtypeStruct((n, n), jnp.float32),
    )(agg_feature, agg_feature.T)

    BN = 512
    ctx = pl.pallas_call(
        _attn_body,
        grid=(node_pad // BN,),
        in_specs=[
            pl.BlockSpec((BN, d), lambda i: (i, 0)),
            pl.BlockSpec((BN, md), lambda i: (i, 0)),
        ],
        out_specs=pl.BlockSpec((BN, d), lambda i: (i, 0)),
        out_shape=jax.ShapeDtypeStruct((node_pad, d), jnp.float32),
    )(agg_pad, nb2)[:n]

    BR = 1024
    attr = pl.pallas_call(
        _attr_body,
        grid=(pl.cdiv(n, BR),),
        in_specs=[
            pl.BlockSpec((BR, node_dim), lambda i: (i, 0)),
            pl.BlockSpec((BR, d), lambda i: (i, 0)),
            pl.BlockSpec((node_dim, node_dim), lambda i: (0, 0)),
            pl.BlockSpec((d, node_dim), lambda i: (0, 0)),
            pl.BlockSpec((1, node_dim), lambda i: (0, 0)),
        ],
        out_specs=pl.BlockSpec((BR, node_dim), lambda i: (i, 0)),
        out_shape=jax.ShapeDtypeStruct((n, node_dim), jnp.float32),
    )(node_feature, ctx, W[:node_dim], W[node_dim:], b.reshape(1, node_dim))

    return (attr, adj)


# speedup vs baseline: 1.2303x; 1.2303x over previous
import functools
import math

import jax
import jax.numpy as jnp
from jax import lax
from jax.experimental import pallas as pl
from jax.experimental.pallas import tpu as pltpu
from jax.experimental.pallas import tpu_sc as plsc

L = 16
NW = 32
NPW = 320
CHUNK = 80
DEG = 32


def _context_body(agg_hbm, nb_hbm, ctx_hbm, aggv, idxv, rows, ctxv, sem):
    cid = lax.axis_index("c")
    sid = lax.axis_index("s")
    wid = sid * 2 + cid
    node_base = wid * NPW
    iota = lax.broadcasted_iota(jnp.int32, (L,), 0)

    for chunk in range(NPW // CHUNK):
        nbase = node_base + chunk * CHUNK
        pltpu.sync_copy(agg_hbm.at[pl.ds(nbase, CHUNK)], aggv)
        pltpu.sync_copy(nb_hbm.at[pl.ds(nbase * DEG, CHUNK * DEG)], idxv)
        pltpu.async_copy(agg_hbm.at[idxv], rows, sem).wait()

        def node_body(t, carry):
            base = t * DEG
            r0 = iota + base
            r1 = iota + (base + L)
            av = aggv[t, :]
            sa = jnp.zeros((L,), jnp.float32)
            sb = jnp.zeros((L,), jnp.float32)
            for dd in range(L):
                a_d = av[dd]
                col = jnp.full((L,), dd, jnp.int32)
                sa = sa + a_d * plsc.load_gather(rows, [r0, col])
                sb = sb + a_d * plsc.load_gather(rows, [r1, col])
            sa = sa * jnp.float32(1.0 / math.sqrt(L))
            sb = sb * jnp.float32(1.0 / math.sqrt(L))
            mx = jnp.maximum(jnp.max(sa), jnp.max(sb))
            ea = jnp.exp(sa - mx)
            eb = jnp.exp(sb - mx)
            total = jnp.sum(ea) + jnp.sum(eb)
            inv = jnp.ones((L,), jnp.float32) / jnp.broadcast_to(total, (L,))
            pa = ea * inv
            pb = eb * inv
            ctx = jnp.zeros((L,), jnp.float32)
            for m in range(L):
                ctx = ctx + pa[m] * rows[base + m, :]
            for m in range(L):
                ctx = ctx + pb[m] * rows[base + L + m, :]
            ctxv[t, :] = ctx
            return carry

        lax.fori_loop(0, CHUNK, node_body, 0)
        pltpu.sync_copy(ctxv, ctx_hbm.at[pl.ds(nbase, CHUNK)])


def _context_sc(agg_pad, nb_pad, node_pad):
    f = functools.partial(
        pl.kernel,
        out_type=jax.ShapeDtypeStruct((node_pad, L), jnp.float32),
        mesh=plsc.VectorSubcoreMesh(core_axis_name="c", subcore_axis_name="s"),
        compiler_params=pltpu.CompilerParams(
            needs_layout_passes=False, use_tc_tiling_on_sc=False),
        scratch_types=[
            pltpu.VMEM((CHUNK, L), jnp.float32),
            pltpu.VMEM((CHUNK * DEG,), jnp.int32),
            pltpu.VMEM((CHUNK * DEG, L), jnp.float32),
            pltpu.VMEM((CHUNK, L), jnp.float32),
            pltpu.SemaphoreType.DMA,
        ],
    )(_context_body)
    return f(agg_pad, nb_pad)


def _adj_body(a_ref, b_ref, o_ref):
    x = lax.dot_general(
        a_ref[...], b_ref[...], (((1,), (1,)), ((), ())),
        preferred_element_type=jnp.float32)
    o_ref[...] = 0.5 * (jnp.tanh(0.5 * x) + 1.0)


def _attr_body(nf_ref, ctx_ref, w1_ref, w2_ref, b_ref, o_ref):
    acc = jnp.dot(nf_ref[...], w1_ref[...], preferred_element_type=jnp.float32)
    acc = acc + jnp.dot(ctx_ref[...], w2_ref[...], preferred_element_type=jnp.float32)
    o_ref[...] = jax.nn.sigmoid(acc + b_ref[...])


def kernel(node_feature, agg_feature, nb_id, W, b):
    n = agg_feature.shape[0]
    d = agg_feature.shape[1]
    node_dim = node_feature.shape[1]
    node_pad = NW * NPW

    agg_pad = jnp.pad(agg_feature, ((0, node_pad - n), (0, 0)))
    nb_pad = jnp.pad(nb_id, (0, node_pad * DEG - nb_id.shape[0]))

    ctx = _context_sc(agg_pad, nb_pad, node_pad)[:n]

    BI, BJ = 512, 2048
    adj = pl.pallas_call(
        _adj_body,
        grid=(pl.cdiv(n, BI), pl.cdiv(n, BJ)),
        in_specs=[
            pl.BlockSpec((BI, d), lambda i, j: (i, 0)),
            pl.BlockSpec((BJ, d), lambda i, j: (j, 0)),
        ],
        out_specs=pl.BlockSpec((BI, BJ), lambda i, j: (i, j)),
        out_shape=jax.ShapeDtypeStruct((n, n), jnp.float32),
    )(agg_feature, agg_feature)

    BR = 1024
    attr = pl.pallas_call(
        _attr_body,
        grid=(pl.cdiv(n, BR),),
        in_specs=[
            pl.BlockSpec((BR, node_dim), lambda i: (i, 0)),
            pl.BlockSpec((BR, d), lambda i: (i, 0)),
            pl.BlockSpec((node_dim, node_dim), lambda i: (0, 0)),
            pl.BlockSpec((d, node_dim), lambda i: (0, 0)),
            pl.BlockSpec((1, node_dim), lambda i: (0, 0)),
        ],
        out_specs=pl.BlockSpec((BR, node_dim), lambda i: (i, 0)),
        out_shape=jax.ShapeDtypeStruct((n, node_dim), jnp.float32),
    )(node_feature, ctx, W[:node_dim], W[node_dim:], b.reshape(1, node_dim))

    return (attr, adj)


# speedup vs baseline: 1.3899x; 1.1297x over previous
import functools
import math

import jax
import jax.numpy as jnp
from jax import lax
from jax.experimental import pallas as pl
from jax.experimental.pallas import tpu as pltpu
from jax.experimental.pallas import tpu_sc as plsc

L = 16
NW = 32
NPW = 320
CHUNK = 80
DEG = 32


def _context_body(agg_hbm, nb_hbm, ctx_hbm, aggv, idxv, rows, ctxv, sem):
    cid = lax.axis_index("c")
    sid = lax.axis_index("s")
    wid = sid * 2 + cid
    node_base = wid * NPW
    iota = lax.broadcasted_iota(jnp.int32, (L,), 0)

    for chunk in range(NPW // CHUNK):
        nbase = node_base + chunk * CHUNK
        pltpu.sync_copy(agg_hbm.at[pl.ds(nbase, CHUNK)], aggv)
        pltpu.sync_copy(nb_hbm.at[pl.ds(nbase * DEG, CHUNK * DEG)], idxv)
        pltpu.async_copy(agg_hbm.at[idxv], rows, sem).wait()

        def node_body(t, carry):
            base = t * DEG
            r0 = iota + base
            r1 = iota + (base + L)
            av = aggv[t, :]
            sa = jnp.zeros((L,), jnp.float32)
            sb = jnp.zeros((L,), jnp.float32)
            for dd in range(L):
                a_d = av[dd]
                col = jnp.full((L,), dd, jnp.int32)
                sa = sa + a_d * plsc.load_gather(rows, [r0, col])
                sb = sb + a_d * plsc.load_gather(rows, [r1, col])
            sa = sa * jnp.float32(1.0 / math.sqrt(L))
            sb = sb * jnp.float32(1.0 / math.sqrt(L))
            mx = jnp.maximum(jnp.max(sa), jnp.max(sb))
            ea = jnp.exp(sa - mx)
            eb = jnp.exp(sb - mx)
            total = jnp.sum(ea) + jnp.sum(eb)
            inv = jnp.ones((L,), jnp.float32) / jnp.broadcast_to(total, (L,))
            pa = ea * inv
            pb = eb * inv
            ctx = jnp.zeros((L,), jnp.float32)
            for m in range(L):
                ctx = ctx + pa[m] * rows[base + m, :]
            for m in range(L):
                ctx = ctx + pb[m] * rows[base + L + m, :]
            ctxv[t, :] = ctx
            return carry

        lax.fori_loop(0, CHUNK, node_body, 0)
        pltpu.sync_copy(ctxv, ctx_hbm.at[pl.ds(nbase, CHUNK)])


def _context_sc(agg_pad, nb_pad, node_pad):
    f = functools.partial(
        pl.kernel,
        out_type=jax.ShapeDtypeStruct((node_pad, L), jnp.float32),
        mesh=plsc.VectorSubcoreMesh(core_axis_name="c", subcore_axis_name="s"),
        compiler_params=pltpu.CompilerParams(
            needs_layout_passes=False, use_tc_tiling_on_sc=False),
        scratch_types=[
            pltpu.VMEM((CHUNK, L), jnp.float32),
            pltpu.VMEM((CHUNK * DEG,), jnp.int32),
            pltpu.VMEM((CHUNK * DEG, L), jnp.float32),
            pltpu.VMEM((CHUNK, L), jnp.float32),
            pltpu.SemaphoreType.DMA,
        ],
    )(_context_body)
    return f(agg_pad, nb_pad)


def _adj_body(a_ref, b_ref, o_ref):
    x = lax.dot_general(
        a_ref[...], b_ref[...], (((1,), (1,)), ((), ())),
        preferred_element_type=jnp.float32)
    o_ref[...] = 0.5 * (jnp.tanh(0.5 * x) + 1.0)


def _attr_body(nf_ref, ctx_ref, w1_ref, w2_ref, b_ref, o_ref):
    acc = jnp.dot(nf_ref[...], w1_ref[...], preferred_element_type=jnp.float32)
    acc = acc + jnp.dot(ctx_ref[...], w2_ref[...], preferred_element_type=jnp.float32)
    o_ref[...] = jax.nn.sigmoid(acc + b_ref[...])


def kernel(node_feature, agg_feature, nb_id, W, b):
    n = agg_feature.shape[0]
    d = agg_feature.shape[1]
    node_dim = node_feature.shape[1]
    node_pad = NW * NPW

    agg_pad = jnp.pad(agg_feature, ((0, node_pad - n), (0, 0)))
    nb_pad = jnp.pad(nb_id, (0, node_pad * DEG - nb_id.shape[0]))

    ctx = _context_sc(agg_pad, nb_pad, node_pad)[:n]

    BI, BJ = 1024, 2048
    adj = pl.pallas_call(
        _adj_body,
        grid=(pl.cdiv(n, BI), pl.cdiv(n, BJ)),
        in_specs=[
            pl.BlockSpec((BI, d), lambda i, j: (i, 0)),
            pl.BlockSpec((BJ, d), lambda i, j: (j, 0)),
        ],
        out_specs=pl.BlockSpec((BI, BJ), lambda i, j: (i, j)),
        out_shape=jax.ShapeDtypeStruct((n, n), jnp.float32),
    )(agg_feature, agg_feature)

    BR = 1024
    attr = pl.pallas_call(
        _attr_body,
        grid=(pl.cdiv(n, BR),),
        in_specs=[
            pl.BlockSpec((BR, node_dim), lambda i: (i, 0)),
            pl.BlockSpec((BR, d), lambda i: (i, 0)),
            pl.BlockSpec((node_dim, node_dim), lambda i: (0, 0)),
            pl.BlockSpec((d, node_dim), lambda i: (0, 0)),
            pl.BlockSpec((1, node_dim), lambda i: (0, 0)),
        ],
        out_specs=pl.BlockSpec((BR, node_dim), lambda i: (i, 0)),
        out_shape=jax.ShapeDtypeStruct((n, node_dim), jnp.float32),
    )(node_feature, ctx, W[:node_dim], W[node_dim:], b.reshape(1, node_dim))

    return (attr, adj)


# speedup vs baseline: 1.4715x; 1.0587x over previous
import functools
import math

import jax
import jax.numpy as jnp
from jax import lax
from jax.experimental import pallas as pl
from jax.experimental.pallas import tpu as pltpu
from jax.experimental.pallas import tpu_sc as plsc

L = 16
NW = 32
NPW = 320
CHUNK = 80
DEG = 32


def _context_body(agg_hbm, nb_hbm, ctx_hbm, aggv, idxv, rows, ctxv, sem):
    cid = lax.axis_index("c")
    sid = lax.axis_index("s")
    wid = sid * 2 + cid
    node_base = wid * NPW
    iota = lax.broadcasted_iota(jnp.int32, (L,), 0)

    for chunk in range(NPW // CHUNK):
        nbase = node_base + chunk * CHUNK
        pltpu.sync_copy(agg_hbm.at[pl.ds(nbase, CHUNK)], aggv)
        pltpu.sync_copy(nb_hbm.at[pl.ds(nbase * DEG, CHUNK * DEG)], idxv)
        pltpu.async_copy(agg_hbm.at[idxv], rows, sem).wait()

        def node_body(t, carry):
            base = t * DEG
            r0 = iota + base
            r1 = iota + (base + L)
            av = aggv[t, :]
            sa = jnp.zeros((L,), jnp.float32)
            sb = jnp.zeros((L,), jnp.float32)
            for dd in range(L):
                a_d = av[dd]
                col = jnp.full((L,), dd, jnp.int32)
                sa = sa + a_d * plsc.load_gather(rows, [r0, col])
                sb = sb + a_d * plsc.load_gather(rows, [r1, col])
            sa = sa * jnp.float32(1.0 / math.sqrt(L))
            sb = sb * jnp.float32(1.0 / math.sqrt(L))
            mx = jnp.maximum(jnp.max(sa), jnp.max(sb))
            ea = jnp.exp(sa - mx)
            eb = jnp.exp(sb - mx)
            total = jnp.sum(ea) + jnp.sum(eb)
            inv = jnp.ones((L,), jnp.float32) / jnp.broadcast_to(total, (L,))
            pa = ea * inv
            pb = eb * inv
            ctx = jnp.zeros((L,), jnp.float32)
            for m in range(L):
                ctx = ctx + pa[m] * rows[base + m, :]
            for m in range(L):
                ctx = ctx + pb[m] * rows[base + L + m, :]
            ctxv[t, :] = ctx
            return carry

        lax.fori_loop(0, CHUNK, node_body, 0)
        pltpu.sync_copy(ctxv, ctx_hbm.at[pl.ds(nbase, CHUNK)])


def _context_sc(agg_pad, nb_pad, node_pad):
    f = functools.partial(
        pl.kernel,
        out_type=jax.ShapeDtypeStruct((node_pad, L), jnp.float32),
        mesh=plsc.VectorSubcoreMesh(core_axis_name="c", subcore_axis_name="s"),
        compiler_params=pltpu.CompilerParams(
            needs_layout_passes=False, use_tc_tiling_on_sc=False),
        scratch_types=[
            pltpu.VMEM((CHUNK, L), jnp.float32),
            pltpu.VMEM((CHUNK * DEG,), jnp.int32),
            pltpu.VMEM((CHUNK * DEG, L), jnp.float32),
            pltpu.VMEM((CHUNK, L), jnp.float32),
            pltpu.SemaphoreType.DMA,
        ],
    )(_context_body)
    return f(agg_pad, nb_pad)


def _adj_body(a_ref, b_ref, o_ref):
    x = lax.dot_general(
        a_ref[...], b_ref[...], (((1,), (1,)), ((), ())),
        preferred_element_type=jnp.float32)
    o_ref[...] = 0.5 * (jnp.tanh(0.5 * x) + 1.0)


def _attr_body(nf_ref, ctx_ref, w1_ref, w2_ref, b_ref, o_ref):
    acc = jnp.dot(nf_ref[...], w1_ref[...], preferred_element_type=jnp.float32)
    acc = acc + jnp.dot(ctx_ref[...], w2_ref[...], preferred_element_type=jnp.float32)
    o_ref[...] = jax.nn.sigmoid(acc + b_ref[...])


def kernel(node_feature, agg_feature, nb_id, W, b):
    n = agg_feature.shape[0]
    d = agg_feature.shape[1]
    node_dim = node_feature.shape[1]
    node_pad = NW * NPW

    agg_pad = jnp.pad(agg_feature, ((0, node_pad - n), (0, 0)))
    nb_pad = jnp.pad(nb_id, (0, node_pad * DEG - nb_id.shape[0]))

    ctx = _context_sc(agg_pad, nb_pad, node_pad)[:n]

    BI, BJ = 2048, 2048
    adj = pl.pallas_call(
        _adj_body,
        grid=(pl.cdiv(n, BI), pl.cdiv(n, BJ)),
        in_specs=[
            pl.BlockSpec((BI, d), lambda i, j: (i, 0)),
            pl.BlockSpec((BJ, d), lambda i, j: (j, 0)),
        ],
        out_specs=pl.BlockSpec((BI, BJ), lambda i, j: (i, j)),
        out_shape=jax.ShapeDtypeStruct((n, n), jnp.float32),
    )(agg_feature, agg_feature)

    BR = 1024
    attr = pl.pallas_call(
        _attr_body,
        grid=(pl.cdiv(n, BR),),
        in_specs=[
            pl.BlockSpec((BR, node_dim), lambda i: (i, 0)),
            pl.BlockSpec((BR, d), lambda i: (i, 0)),
            pl.BlockSpec((node_dim, node_dim), lambda i: (0, 0)),
            pl.BlockSpec((d, node_dim), lambda i: (0, 0)),
            pl.BlockSpec((1, node_dim), lambda i: (0, 0)),
        ],
        out_specs=pl.BlockSpec((BR, node_dim), lambda i: (i, 0)),
        out_shape=jax.ShapeDtypeStruct((n, node_dim), jnp.float32),
    )(node_feature, ctx, W[:node_dim], W[node_dim:], b.reshape(1, node_dim))

    return (attr, adj)
